# baseline (speedup 1.0000x reference)
import functools

import jax
import jax.numpy as jnp
from jax import lax
from jax.experimental import pallas as pl
from jax.experimental.pallas import tpu as pltpu
from jax.experimental.pallas import tpu_sc as plsc

_NC = 2
_NS = 16
_NW = _NC * _NS
_NBUF = 4
_L = 16
_BT = 128
_DT = 8


def _embed_kernel(B, T, D):
    assert B // _BT == _NW
    mesh = plsc.VectorSubcoreMesh(core_axis_name="c", subcore_axis_name="s")

    @functools.partial(
        pl.kernel,
        out_type=jax.ShapeDtypeStruct((T, D // _DT, _NW, _DT, _BT),
                                      jnp.float32),
        mesh=mesh,
        compiler_params=pltpu.CompilerParams(
            use_tc_tiling_on_sc=False, needs_layout_passes=False),
        scratch_types=[
            pltpu.VMEM((T, _BT), jnp.int32),
            pltpu.VMEM((T, D), jnp.float32),
            [pltpu.VMEM((_BT, D), jnp.float32) for _ in range(_NBUF)],
            [pltpu.VMEM((D // _DT, _DT, _BT), jnp.float32)
             for _ in range(_NBUF)],
            [pltpu.SemaphoreType.DMA for _ in range(_NBUF)],
            [pltpu.SemaphoreType.DMA for _ in range(_NBUF)],
        ],
    )
    def k(xt_hbm, tab_hbm, pos_hbm, out_hbm,
          idx_v, pos_v, bufs, obufs, sgs, sos):
        sid = lax.axis_index("s")
        wid = sid * _NC + lax.axis_index("c")
        b0 = wid * _BT

        pltpu.sync_copy(xt_hbm.at[:, pl.ds(b0, _BT)], idx_v)
        pltpu.sync_copy(pos_hbm, pos_v)

        def issue_gather(t, j):
            pltpu.async_copy(tab_hbm.at[idx_v.at[t]], bufs[j], sgs[j])

        def drain_gather(j):
            pltpu.make_async_copy(
                tab_hbm.at[pl.ds(0, _BT)], bufs[j], sgs[j]).wait()

        def issue_out(t, j):
            pltpu.async_copy(obufs[j], out_hbm.at[t, :, wid], sos[j])

        def drain_out(j):
            pltpu.make_async_copy(
                out_hbm.at[0, :, 0], obufs[j], sos[j]).wait()

        iota = lax.iota(jnp.int32, _L)
        zf = jnp.zeros((_L,), jnp.float32)

        def transpose_add(t, j):
            buf, obuf = bufs[j], obufs[j]
            posg = [pos_v[t, pl.ds(_L * g, _L)] for g in range(D // _L)]

            @plsc.parallel_loop(0, _BT // _L, unroll=2)
            def _tr(m):
                rows = iota + _L * m
                for d in range(D):
                    cols = jnp.full((_L,), d, jnp.int32)
                    g = plsc.load_gather(buf, [rows, cols])
                    p = zf + posg[d // _L][d % _L]
                    obuf[d // _DT, d % _DT, pl.ds(_L * m, _L)] = g + p

        for j in range(_NBUF):
            issue_gather(j, j)

        def body(g, carry):
            for j in range(_NBUF):
                t = g * _NBUF + j
                drain_gather(j)

                @pl.when(g > 0)
                def _d():
                    drain_out(j)

                transpose_add(t, j)

                @pl.when(t + _NBUF < T)
                def _g():
                    issue_gather(t + _NBUF, j)

                issue_out(t, j)
            return carry

        lax.fori_loop(0, T // _NBUF, body, 0)
        for j in range(_NBUF):
            drain_out(j)

    return k


def kernel(x, token_embed_tab, positional_embeddings):
    B, T = x.shape
    D = token_embed_tab.shape[1]
    out5 = _embed_kernel(B, T, D)(
        x.T, token_embed_tab, positional_embeddings)
    out = out5.transpose(2, 4, 0, 1, 3).reshape(B, T, D)
    return out


# speedup vs baseline: 1.2243x; 1.2243x over previous
import functools

import jax
import jax.numpy as jnp
from jax import lax
from jax.experimental import pallas as pl
from jax.experimental.pallas import tpu as pltpu
from jax.experimental.pallas import tpu_sc as plsc

_NC = 2
_NS = 16
_NW = _NC * _NS
_NBUF = 4
_L = 16
_STEP = 64


def _embed_kernel(N, T, D):
    per_w = N // _NW
    n_steps = per_w // _STEP
    dv = D // _L
    mesh = plsc.VectorSubcoreMesh(core_axis_name="c", subcore_axis_name="s")

    @functools.partial(
        pl.kernel,
        out_type=jax.ShapeDtypeStruct((N, D), jnp.float32),
        mesh=mesh,
        compiler_params=pltpu.CompilerParams(use_tc_tiling_on_sc=False),
        scratch_types=[
            pltpu.VMEM((per_w,), jnp.int32),
            pltpu.VMEM((2 * T, D), jnp.float32),
            [pltpu.VMEM((_STEP, 2 * D), jnp.float32) for _ in range(_NBUF)],
            [pltpu.VMEM((_STEP, D), jnp.float32) for _ in range(_NBUF)],
            [pltpu.SemaphoreType.DMA for _ in range(_NBUF)],
            [pltpu.SemaphoreType.DMA for _ in range(_NBUF)],
        ],
    )
    def k(x_hbm, tabp_hbm, pos2_hbm, out_hbm,
          idx_v, pos_v, bufs, obufs, sgs, sos):
        sid = lax.axis_index("s")
        wid = sid * _NC + lax.axis_index("c")
        base = wid * per_w

        pltpu.sync_copy(x_hbm.at[pl.ds(base, per_w)], idx_v)
        pltpu.sync_copy(pos2_hbm, pos_v)

        def issue_gather(s, j):
            pltpu.async_copy(
                tabp_hbm.at[idx_v.at[pl.ds(s * _STEP, _STEP)]],
                bufs[j], sgs[j])

        def drain_gather(j):
            pltpu.make_async_copy(
                tabp_hbm.at[pl.ds(0, _STEP)], bufs[j], sgs[j]).wait()

        def issue_out(s, j):
            pltpu.async_copy(
                obufs[j], out_hbm.at[pl.ds(base + s * _STEP, _STEP)], sos[j])

        def drain_out(j):
            pltpu.make_async_copy(
                out_hbm.at[pl.ds(0, _STEP)], obufs[j], sos[j]).wait()

        for j in range(_NBUF):
            issue_gather(j, j)

        def body(g, carry):
            for j in range(_NBUF):
                s = g * _NBUF + j
                off = lax.rem(s * _STEP, T)
                drain_gather(j)

                @pl.when(g > 0)
                def _d():
                    drain_out(j)

                buf, obuf = bufs[j], obufs[j]

                @plsc.parallel_loop(0, _STEP, unroll=4)
                def _add(r):
                    for kk in range(dv):
                        sl = pl.ds(_L * kk, _L)
                        obuf[r, sl] = buf[r, sl] + pos_v[off + r, sl]

                @pl.when(s + _NBUF < n_steps)
                def _g():
                    issue_gather(s + _NBUF, j)

                issue_out(s, j)
            return carry

        lax.fori_loop(0, n_steps // _NBUF, body, 0)
        for j in range(_NBUF):
            drain_out(j)

    return k


def kernel(x, token_embed_tab, positional_embeddings):
    B, T = x.shape
    D = token_embed_tab.shape[1]
    N = B * T
    pos2 = jnp.concatenate([positional_embeddings, positional_embeddings], 0)
    tabp = jnp.pad(token_embed_tab, ((0, 0), (0, D)))
    out = _embed_kernel(N, T, D)(x.reshape(N), tabp, pos2)
    return out.reshape(B, T, D)


# speedup vs baseline: 2.0337x; 1.6612x over previous
import functools

import jax
import jax.numpy as jnp
from jax import lax
from jax.experimental import pallas as pl
from jax.experimental.pallas import tpu as pltpu
from jax.experimental.pallas import tpu_sc as plsc

_NC = 2
_NS = 16
_NW = _NC * _NS
_NBUF = 4
_L = 16
_BT = 128
_DT = 8
_PAD = 131


def _embed_kernel(B, T, D):
    assert B // _BT == _NW
    dv = D // _L
    mesh = plsc.VectorSubcoreMesh(core_axis_name="c", subcore_axis_name="s")

    @functools.partial(
        pl.kernel,
        out_type=jax.ShapeDtypeStruct((T, D // _DT, _NW, _DT, _BT),
                                      jnp.float32),
        mesh=mesh,
        compiler_params=pltpu.CompilerParams(
            use_tc_tiling_on_sc=False, needs_layout_passes=False),
        scratch_types=[
            pltpu.VMEM((T, _BT), jnp.int32),
            pltpu.VMEM((T, D), jnp.float32),
            [pltpu.VMEM((_BT, D), jnp.float32) for _ in range(_NBUF)],
            [pltpu.VMEM((D // _DT, _DT, _PAD), jnp.float32)
             for _ in range(_NBUF)],
            [pltpu.SemaphoreType.DMA for _ in range(_NBUF)],
            [pltpu.SemaphoreType.DMA for _ in range(_NBUF)],
        ],
    )
    def k(xt_hbm, tab_hbm, pos_hbm, out_hbm,
          idx_v, pos_v, bufs, obufs, sgs, sos):
        sid = lax.axis_index("s")
        wid = sid * _NC + lax.axis_index("c")
        b0 = wid * _BT

        pltpu.sync_copy(xt_hbm.at[:, pl.ds(b0, _BT)], idx_v)
        pltpu.sync_copy(pos_hbm, pos_v)

        def issue_gather(t, j):
            pltpu.async_copy(tab_hbm.at[idx_v.at[t]], bufs[j], sgs[j])

        def drain_gather(j):
            pltpu.make_async_copy(
                tab_hbm.at[pl.ds(0, _BT)], bufs[j], sgs[j]).wait()

        def issue_out(t, j):
            pltpu.async_copy(
                obufs[j].at[:, :, pl.ds(0, _BT)],
                out_hbm.at[t, :, wid], sos[j])

        def drain_out(j):
            pltpu.make_async_copy(
                out_hbm.at[0, :, 0],
                obufs[j].at[:, :, pl.ds(0, _BT)], sos[j]).wait()

        iota = lax.iota(jnp.int32, _L)
        zero = iota * 0
        dhi = [(iota + _L * kk) >> 3 for kk in range(dv)]
        dlo = [(iota + _L * kk) & 7 for kk in range(dv)]

        def transpose_add(t, j):
            buf, obuf = bufs[j], obufs[j]
            posg = [pos_v[t, pl.ds(_L * kk, _L)] for kk in range(dv)]

            @plsc.parallel_loop(0, _BT, unroll=2)
            def _tr(b):
                colv = zero + b
                for kk in range(dv):
                    v = buf[b, pl.ds(_L * kk, _L)] + posg[kk]
                    plsc.store_scatter(obuf, [dhi[kk], dlo[kk], colv], v)

        for j in range(_NBUF):
            issue_gather(j, j)

        def body(g, carry):
            for j in range(_NBUF):
                t = g * _NBUF + j
                drain_gather(j)

                @pl.when(g > 0)
                def _d():
                    drain_out(j)

                transpose_add(t, j)

                @pl.when(t + _NBUF < T)
                def _g():
                    issue_gather(t + _NBUF, j)

                issue_out(t, j)
            return carry

        lax.fori_loop(0, T // _NBUF, body, 0)
        for j in range(_NBUF):
            drain_out(j)

    return k


def kernel(x, token_embed_tab, positional_embeddings):
    B, T = x.shape
    D = token_embed_tab.shape[1]
    out5 = _embed_kernel(B, T, D)(
        x.T, token_embed_tab, positional_embeddings)
    return out5.transpose(2, 4, 0, 1, 3).reshape(B, T, D)
